# baseline (speedup 1.0000x reference)
import functools

import jax
import jax.numpy as jnp
from jax import lax
from jax.experimental import pallas as pl
from jax.experimental.pallas import tpu as pltpu
from jax.experimental.pallas import tpu_sc as plsc

_N = 320000
_DI = 125
_DP = 3
_DO = 128
_NSEG = 16
_B = 16000
_G = _B // 8
_NBLK = _N // _B
_IMAX = jnp.iinfo(jnp.int32).max

_NT = 16
_CH = _N // _NT
_L = 16


def _seg_kernel(batch_smem, x_ref, pos_ref, w1_ref, w2_ref, b_ref,
                out_ref, h_ref):
    i = pl.program_id(0)

    @pl.when(i == 0)
    def _init():
        out_ref[...] = jnp.full((_NSEG, _DO), -jnp.inf, jnp.float32)

    h_ref[...] = (
        jnp.dot(x_ref[...], w1_ref[...], preferred_element_type=jnp.float32)
        + lax.dot_general(pos_ref[...], w2_ref[...],
                          dimension_numbers=(((0,), (0,)), ((), ())),
                          preferred_element_type=jnp.float32))

    g8 = jnp.max(h_ref[...].reshape(_G, 8, _DO), axis=1)

    first = batch_smem[0, 0, 0]
    last = batch_smem[0, 0, _B - 1]

    seg_rows = lax.broadcasted_iota(jnp.int32, (_NSEG, _DO), 0)

    @pl.when(first == last)
    def _fast():
        mx = jnp.max(g8, axis=0, keepdims=True)
        sel = seg_rows == first
        out_ref[...] = jnp.where(sel, jnp.maximum(out_ref[...], mx), out_ref[...])

    @pl.when(first != last)
    def _slow():
        grp_start = lax.broadcasted_iota(jnp.int32, (_G, 1), 0) * 8
        row8 = lax.broadcasted_iota(jnp.int32, (8, 1), 0)

        def lower_bound(v):
            def cond(c):
                return c[0] < c[1]

            def step(c):
                lo, hi = c
                mid = (lo + hi) // 2
                go_right = batch_smem[0, 0, mid] < v
                return (jnp.where(go_right, mid + 1, lo),
                        jnp.where(go_right, hi, mid))

            return lax.while_loop(cond, step, (0, _B))[0]

        def body(s, carry):
            start = lower_bound(s)
            end = lower_bound(s + 1) - 1
            gin = (grp_start >= start) & (grp_start + 7 <= end)
            mx = jnp.max(jnp.where(gin, g8, -jnp.inf), axis=0, keepdims=True)
            for q in (jnp.clip(start // 8, 0, _G - 1), jnp.clip(end // 8, 0, _G - 1)):
                rows = h_ref[pl.ds(q * 8, 8), :]
                rid = q * 8 + row8
                pm = (rid >= start) & (rid <= end)
                mx = jnp.maximum(mx, jnp.max(jnp.where(pm, rows, -jnp.inf),
                                             axis=0, keepdims=True))
            sel = seg_rows == s
            out_ref[...] = jnp.where(sel, jnp.maximum(out_ref[...], mx), out_ref[...])
            return carry

        lax.fori_loop(first, last + 1, body, 0)

    @pl.when(i == _NBLK - 1)
    def _bias():
        out_ref[...] = out_ref[...] + b_ref[...]


def _sc_qidx_kernel(batch_hbm, pos_hbm, pos_out_hbm, batch_out_hbm,
                    buf, qloc, comb, prow, bout, shared, sem):
    cid = lax.axis_index("c")
    sid = lax.axis_index("s")
    lanes = lax.iota(jnp.int32, _L)

    @pl.when(cid == 0)
    def _scan():
        base = sid * _CH
        pltpu.sync_copy(batch_hbm.at[pl.ds(base, _CH)], buf.at[pl.ds(0, _CH)])

        n_steps = max(1, (_CH - 1).bit_length())

        def lower_bound(v):
            def step(_, c):
                lo, hi = c
                mid = (lo + hi) // 2
                go_right = buf[pl.ds(mid, _L)][0] < v
                return (jnp.where(go_right, mid + 1, lo),
                        jnp.where(go_right, hi, mid))

            return lax.fori_loop(0, n_steps, step, (0, _CH))[0]

        m = jnp.full((_L,), _IMAX, jnp.int32)
        for s in range(_NSEG):
            j = lower_bound(s)
            present = (j < _CH) & (buf[pl.ds(jnp.minimum(j, _CH - 1), _L)][0] == s)
            cand = jnp.where(present, base + j, _IMAX)
            m = jnp.where(lanes == s, cand, m)
        qloc[...] = m
        pltpu.sync_copy(qloc, shared.at[pl.ds(sid * _NSEG, _NSEG)])

    plsc.subcore_barrier()

    @pl.when((cid == 0) & (sid == 0))
    def _final():
        pltpu.sync_copy(shared, comb)
        m = comb[pl.ds(0, _NSEG)]
        for w in range(1, _NT):
            m = jnp.minimum(m, comb[pl.ds(w * _NSEG, _NSEG)])
        mc = jnp.minimum(m, _N - 1)
        copies = []
        for s in range(_NSEG):
            copies.append(pltpu.async_copy(
                pos_hbm.at[pl.ds(mc[s], 1)], prow.at[pl.ds(s, 1)], sem))
        present = m != _IMAX
        maxid = jnp.int32(-1)
        for s in range(_NSEG):
            maxid = jnp.where(m[s] != _IMAX, jnp.int32(s), maxid)
        bout[...] = jnp.where(present, lanes, maxid)
        for c in copies:
            c.wait()
        pltpu.sync_copy(prow, pos_out_hbm)
        pltpu.sync_copy(bout, batch_out_hbm)


_sc_qidx = functools.partial(
    pl.kernel,
    mesh=plsc.VectorSubcoreMesh(core_axis_name="c", subcore_axis_name="s"),
    out_type=[
        jax.ShapeDtypeStruct((_NSEG, _DP), jnp.float32),
        jax.ShapeDtypeStruct((_NSEG,), jnp.int32),
    ],
    scratch_types=[
        pltpu.VMEM((_CH + _L,), jnp.int32),
        pltpu.VMEM((_NSEG,), jnp.int32),
        pltpu.VMEM((_NT * _NSEG,), jnp.int32),
        pltpu.VMEM((_NSEG, _DP), jnp.float32),
        pltpu.VMEM((_NSEG,), jnp.int32),
        pltpu.VMEM_SHARED((_NT * _NSEG,), jnp.int32),
        pltpu.SemaphoreType.DMA,
    ],
)(_sc_qidx_kernel)


def kernel(x, pos, batch, W, b):
    batch3 = batch.reshape(_NBLK, 1, _B)
    pos_t = pos.T
    w1 = W[:_DI]
    w2 = W[_DI:]
    b2 = b.reshape(1, _DO)

    pos_out, batch_out = _sc_qidx(batch, pos)

    pooled, = pl.pallas_call(
        _seg_kernel,
        grid=(_NBLK,),
        in_specs=[
            pl.BlockSpec((1, 1, _B), lambda i: (i, 0, 0), memory_space=pltpu.SMEM),
            pl.BlockSpec((_B, _DI), lambda i: (i, 0)),
            pl.BlockSpec((_DP, _B), lambda i: (0, i)),
            pl.BlockSpec((_DI, _DO), lambda i: (0, 0)),
            pl.BlockSpec((_DP, _DO), lambda i: (0, 0)),
            pl.BlockSpec((1, _DO), lambda i: (0, 0)),
        ],
        out_specs=[
            pl.BlockSpec((_NSEG, _DO), lambda i: (0, 0)),
        ],
        out_shape=[
            jax.ShapeDtypeStruct((_NSEG, _DO), jnp.float32),
        ],
        scratch_shapes=[pltpu.VMEM((_B, _DO), jnp.float32)],
        compiler_params=pltpu.CompilerParams(
            dimension_semantics=("arbitrary",),
        ),
    )(batch3, x, pos_t, w1, w2, b2)

    return pooled, pos_out, batch_out


# speedup vs baseline: 1.8660x; 1.8660x over previous
import jax
import jax.numpy as jnp
from jax import lax
from jax.experimental import pallas as pl
from jax.experimental.pallas import tpu as pltpu

_N = 320000
_DI = 125
_DP = 3
_DO = 128
_NSEG = 16
_B = 16000
_G = _B // 8
_NBLK = _N // _B
_IMAX = jnp.iinfo(jnp.int32).max


def _seg_kernel(batch_smem, x_ref, pos_ref, w1_ref, w2_ref, b_ref,
                out_ref, qidx_ref, h_ref):
    i = pl.program_id(0)

    @pl.when(i == 0)
    def _init():
        out_ref[...] = jnp.full((_NSEG, _DO), -jnp.inf, jnp.float32)
        qidx_ref[...] = jnp.full((_NSEG, _DO), _IMAX, jnp.int32)

    h_ref[...] = (
        jnp.dot(x_ref[...], w1_ref[...], preferred_element_type=jnp.float32)
        + lax.dot_general(pos_ref[...], w2_ref[...],
                          dimension_numbers=(((0,), (0,)), ((), ())),
                          preferred_element_type=jnp.float32))

    g8 = jnp.max(h_ref[...].reshape(_G, 8, _DO), axis=1)

    first = batch_smem[0, 0, 0]
    last = batch_smem[0, 0, _B - 1]
    base = i * _B

    seg_rows = lax.broadcasted_iota(jnp.int32, (_NSEG, _DO), 0)

    @pl.when(first == last)
    def _fast():
        mx = jnp.max(g8, axis=0, keepdims=True)
        sel = seg_rows == first
        out_ref[...] = jnp.where(sel, jnp.maximum(out_ref[...], mx), out_ref[...])
        qidx_ref[...] = jnp.where(sel, jnp.minimum(qidx_ref[...], base), qidx_ref[...])

    @pl.when(first != last)
    def _slow():
        grp_start = lax.broadcasted_iota(jnp.int32, (_G, 1), 0) * 8
        row8 = lax.broadcasted_iota(jnp.int32, (8, 1), 0)

        def lower_bound(v):
            def cond(c):
                return c[0] < c[1]

            def step(c):
                lo, hi = c
                mid = (lo + hi) // 2
                go_right = batch_smem[0, 0, mid] < v
                return (jnp.where(go_right, mid + 1, lo),
                        jnp.where(go_right, hi, mid))

            return lax.while_loop(cond, step, (0, _B))[0]

        def body(s, carry):
            start = lower_bound(s)
            end = lower_bound(s + 1) - 1
            gin = (grp_start >= start) & (grp_start + 7 <= end)
            mx = jnp.max(jnp.where(gin, g8, -jnp.inf), axis=0, keepdims=True)
            for q in (jnp.clip(start // 8, 0, _G - 1), jnp.clip(end // 8, 0, _G - 1)):
                rows = h_ref[pl.ds(q * 8, 8), :]
                rid = q * 8 + row8
                pm = (rid >= start) & (rid <= end)
                mx = jnp.maximum(mx, jnp.max(jnp.where(pm, rows, -jnp.inf),
                                             axis=0, keepdims=True))
            sel = seg_rows == s
            cand = jnp.where(start <= end, base + start, _IMAX)
            out_ref[...] = jnp.where(sel, jnp.maximum(out_ref[...], mx), out_ref[...])
            qidx_ref[...] = jnp.where(sel, jnp.minimum(qidx_ref[...], cand), qidx_ref[...])
            return carry

        lax.fori_loop(first, last + 1, body, 0)

    @pl.when(i == _NBLK - 1)
    def _bias():
        out_ref[...] = out_ref[...] + b_ref[...]


def kernel(x, pos, batch, W, b):
    batch3 = batch.reshape(_NBLK, 1, _B)
    pos_t = pos.T
    w1 = W[:_DI]
    w2 = W[_DI:]
    b2 = b.reshape(1, _DO)

    pooled, qidx2 = pl.pallas_call(
        _seg_kernel,
        grid=(_NBLK,),
        in_specs=[
            pl.BlockSpec((1, 1, _B), lambda i: (i, 0, 0), memory_space=pltpu.SMEM),
            pl.BlockSpec((_B, _DI), lambda i: (i, 0)),
            pl.BlockSpec((_DP, _B), lambda i: (0, i)),
            pl.BlockSpec((_DI, _DO), lambda i: (0, 0)),
            pl.BlockSpec((_DP, _DO), lambda i: (0, 0)),
            pl.BlockSpec((1, _DO), lambda i: (0, 0)),
        ],
        out_specs=[
            pl.BlockSpec((_NSEG, _DO), lambda i: (0, 0)),
            pl.BlockSpec((_NSEG, _DO), lambda i: (0, 0)),
        ],
        out_shape=[
            jax.ShapeDtypeStruct((_NSEG, _DO), jnp.float32),
            jax.ShapeDtypeStruct((_NSEG, _DO), jnp.int32),
        ],
        scratch_shapes=[pltpu.VMEM((_B, _DO), jnp.float32)],
        compiler_params=pltpu.CompilerParams(
            dimension_semantics=("arbitrary",),
        ),
    )(batch3, x, pos_t, w1, w2, b2)

    qidx = qidx2[:, 0]
    return pooled, pos[qidx], batch[qidx]
